# Initial kernel scaffold; baseline (speedup 1.0000x reference)
import jax
import jax.numpy as jnp
from jax.experimental import pallas as pl


def kernel(patches, forward_indexes, mask_token):
    raise NotImplementedError("write your pallas kernel here")


# speedup vs baseline: 7.7559x; 7.7559x over previous
import functools

import jax
import jax.numpy as jnp
from jax import lax
from jax.experimental import pallas as pl
from jax.experimental.pallas import tpu as pltpu
from jax.experimental.pallas import tpu_sc as plsc

_NC = 2
_NS = 16
_NW = _NC * _NS
_L = 16

_RATIO = 0.75


@functools.lru_cache(maxsize=None)
def _build_sc_call(T, B, C):
    R = int(T * (1 - _RATIO))
    G = R * B
    F = (T - R) * B
    GPW = G // _NW
    ROWS = 64
    NG = GPW // ROWS
    FILLROWS = 2 * ROWS
    FPW = F // _NW
    NF = FPW // FILLROWS
    BPW = B // _NW
    NT = T // _L
    assert G % _NW == 0 and GPW % ROWS == 0 and F % _NW == 0
    assert FPW % FILLROWS == 0 and B % _NW == 0 and T % _L == 0

    mesh = plsc.VectorSubcoreMesh(
        core_axis_name="c", subcore_axis_name="s",
        num_cores=_NC, num_subcores=_NS)

    def body(patches_hbm, fiflat_hbm, fit_hbm, fill_hbm,
             kept_hbm, backt_hbm,
             fi_v, idx_v, rows_v, fib_v, back_v, sem0, sem1):
        wid = lax.axis_index("s") * _NC + lax.axis_index("c")
        base = wid * GPW

        pltpu.sync_copy(fiflat_hbm.at[pl.ds(base, GPW)], fi_v)

        def cidx(g, _):
            for j in range(ROWS // _L):
                off = g * ROWS + j * _L
                fi16 = fi_v[pl.ds(off, _L)]
                r = base + off + lax.iota(jnp.int32, _L)
                idx_v[g, pl.ds(j * _L, _L)] = fi16 * B + lax.rem(r, B)
            return 0
        lax.fori_loop(0, NG, cidx, 0)

        def start(g, buf, sem):
            return pltpu.async_copy(
                patches_hbm.at[idx_v.at[g]],
                rows_v.at[pl.ds(buf * ROWS, ROWS)], sem)

        def drain_write(g, buf, sem):
            pltpu.make_async_copy(
                patches_hbm.at[idx_v.at[g]],
                rows_v.at[pl.ds(buf * ROWS, ROWS)], sem).wait()
            pltpu.sync_copy(rows_v.at[pl.ds(buf * ROWS, ROWS)],
                            kept_hbm.at[pl.ds(base + g * ROWS, ROWS)])

        start(0, 0, sem0)

        def gloop(h, _):
            g0 = 2 * h
            g1 = g0 + 1
            start(g1, 1, sem1)
            drain_write(g0, 0, sem0)

            @pl.when(g1 + 1 < NG)
            def _():
                start(g1 + 1, 0, sem0)
            drain_write(g1, 1, sem1)
            return 0
        lax.fori_loop(0, NG // 2, gloop, 0)

        b0 = wid * BPW
        pltpu.sync_copy(fit_hbm.at[pl.ds(b0, BPW)], fib_v)
        lanes = lax.iota(jnp.int32, _L)
        for bl in range(BPW):
            row = jnp.full((_L,), bl, jnp.int32)

            def scat(k, _, bl=bl, row=row):
                j0 = k * _L
                cols = fib_v[bl, pl.ds(j0, _L)]
                plsc.store_scatter(back_v, [row, cols], j0 + lanes)
                return 0
            lax.fori_loop(0, NT, scat, 0)
        pltpu.sync_copy(back_v, backt_hbm.at[pl.ds(b0, BPW)])

        pltpu.sync_copy(fill_hbm, rows_v)
        fbase = G + wid * FPW

        def floop(t, _):
            pltpu.sync_copy(
                rows_v, kept_hbm.at[pl.ds(fbase + t * FILLROWS, FILLROWS)])
            return 0
        lax.fori_loop(0, NF, floop, 0)

    call = pl.kernel(
        body,
        out_type=(jax.ShapeDtypeStruct((T * B, C), jnp.float32),
                  jax.ShapeDtypeStruct((B, T), jnp.int32)),
        mesh=mesh,
        scratch_types=(
            pltpu.VMEM((GPW,), jnp.int32),
            pltpu.VMEM((NG, ROWS), jnp.int32),
            pltpu.VMEM((FILLROWS, C), jnp.float32),
            pltpu.VMEM((BPW, T), jnp.int32),
            pltpu.VMEM((BPW, T), jnp.int32),
            pltpu.SemaphoreType.DMA,
            pltpu.SemaphoreType.DMA,
        ),
        compiler_params=pltpu.CompilerParams(use_tc_tiling_on_sc=False,
                                             needs_layout_passes=False),
    )
    return call, FILLROWS


def kernel(patches, forward_indexes, mask_token):
    T, B, C = patches.shape
    call, fillrows = _build_sc_call(T, B, C)
    fi = forward_indexes.astype(jnp.int32)
    patches_flat = patches.reshape(T * B, C)
    fi_flat = fi.reshape(T * B)
    fi_t = fi.T
    fill = jnp.broadcast_to(mask_token.reshape(1, C), (fillrows, C))
    kept_flat, backward_t = call(patches_flat, fi_flat, fi_t, fill)
    return kept_flat.reshape(T, B, C), forward_indexes, backward_t.T


# speedup vs baseline: 38.0952x; 4.9118x over previous
import functools

import jax
import jax.numpy as jnp
from jax import lax
from jax.experimental import pallas as pl
from jax.experimental.pallas import tpu as pltpu
from jax.experimental.pallas import tpu_sc as plsc

_NC = 2
_NS = 16
_NW = _NC * _NS
_L = 16

_RATIO = 0.75


@functools.lru_cache(maxsize=None)
def _build_sc_call(T, B, C):
    R = int(T * (1 - _RATIO))
    G = R * B
    F = (T - R) * B
    GPW = G // _NW
    ROWS = 64
    NG = GPW // ROWS
    FILLROWS = 2 * ROWS
    FPW = F // _NW
    NF = FPW // FILLROWS
    BPW = B // _NW
    NT = T // _L
    assert G % _NW == 0 and GPW % ROWS == 0 and F % _NW == 0
    assert FPW % FILLROWS == 0 and B % _NW == 0 and T % _L == 0

    mesh = plsc.VectorSubcoreMesh(
        core_axis_name="c", subcore_axis_name="s",
        num_cores=_NC, num_subcores=_NS)

    def body(patches_hbm, fiflat_hbm, fit_hbm, fill_hbm,
             kept_hbm, backt_hbm,
             fi_v, idx_v, rows_v, fib_v, back_v, sem0, sem1):
        wid = lax.axis_index("s") * _NC + lax.axis_index("c")
        base = wid * GPW

        pltpu.sync_copy(fiflat_hbm.at[pl.ds(base, GPW)], fi_v)

        def cidx(g, _):
            for j in range(ROWS // _L):
                off = g * ROWS + j * _L
                fi16 = fi_v[pl.ds(off, _L)]
                r = base + off + lax.iota(jnp.int32, _L)
                idx_v[g, pl.ds(j * _L, _L)] = fi16 * B + lax.rem(r, B)
            return 0
        lax.fori_loop(0, NG, cidx, 0)

        def start(g, buf, sem):
            return pltpu.async_copy(
                patches_hbm.at[idx_v.at[g]],
                rows_v.at[pl.ds(buf * ROWS, ROWS)], sem)

        def drain_write(g, buf, sem):
            pltpu.make_async_copy(
                patches_hbm.at[idx_v.at[g]],
                rows_v.at[pl.ds(buf * ROWS, ROWS)], sem).wait()
            pltpu.sync_copy(rows_v.at[pl.ds(buf * ROWS, ROWS)],
                            kept_hbm.at[pl.ds(base + g * ROWS, ROWS)])

        start(0, 0, sem0)

        def gloop(h, _):
            g0 = 2 * h
            g1 = g0 + 1
            start(g1, 1, sem1)
            drain_write(g0, 0, sem0)

            @pl.when(g1 + 1 < NG)
            def _():
                start(g1 + 1, 0, sem0)
            drain_write(g1, 1, sem1)
            return 0
        lax.fori_loop(0, NG // 2, gloop, 0)

        b0 = wid * BPW
        pltpu.sync_copy(fit_hbm.at[pl.ds(b0, BPW)], fib_v)
        lanes = lax.iota(jnp.int32, _L)
        for bl in range(BPW):
            row = jnp.full((_L,), bl, jnp.int32)

            def scat(k, _, bl=bl, row=row):
                j0 = k * _L
                cols = fib_v[bl, pl.ds(j0, _L)]
                plsc.store_scatter(back_v, [row, cols], j0 + lanes)
                return 0
            lax.fori_loop(0, NT, scat, 0)
        pltpu.sync_copy(back_v, backt_hbm.at[pl.ds(b0, BPW)])

        pltpu.sync_copy(fill_hbm, rows_v)
        fbase = G + wid * FPW

        def floop(t, _):
            pltpu.sync_copy(
                rows_v, kept_hbm.at[pl.ds(fbase + t * FILLROWS, FILLROWS)])
            return 0
        lax.fori_loop(0, NF, floop, 0)

    call = pl.kernel(
        body,
        out_type=(jax.ShapeDtypeStruct((T * B, C), jnp.float32),
                  jax.ShapeDtypeStruct((B, T), jnp.int32)),
        mesh=mesh,
        scratch_types=(
            pltpu.VMEM((GPW,), jnp.int32),
            pltpu.VMEM((NG, ROWS), jnp.int32),
            pltpu.VMEM((FILLROWS, C), jnp.float32),
            pltpu.VMEM((BPW, T), jnp.int32),
            pltpu.VMEM((BPW, T), jnp.int32),
            pltpu.SemaphoreType.DMA,
            pltpu.SemaphoreType.DMA,
        ),
        compiler_params=pltpu.CompilerParams(use_tc_tiling_on_sc=True,
                                             needs_layout_passes=False),
    )
    return call, FILLROWS


def kernel(patches, forward_indexes, mask_token):
    T, B, C = patches.shape
    call, fillrows = _build_sc_call(T, B, C)
    fi = forward_indexes.astype(jnp.int32)
    patches_flat = patches.reshape(T * B, C)
    fi_flat = fi.reshape(T * B)
    fi_t = fi.T
    fill = jnp.broadcast_to(mask_token.reshape(1, C), (fillrows, C))
    kept_flat, backward_t = call(patches_flat, fi_flat, fi_t, fill)
    return kept_flat.reshape(T, B, C), forward_indexes, backward_t.T
